# baseline (speedup 1.0000x reference)
import functools
import jax
import jax.numpy as jnp
from jax import lax
from jax.experimental import pallas as pl
from jax.experimental.pallas import tpu as pltpu
from jax.experimental.pallas import tpu_sc as plsc

_RADIUS = 0.1
_K = 32
_QT = 128
_MT = 256
_NW = 32


_CH = 2048
_T = 8


def _lexswap(av, ai, bv, bi):
    sw = (bv < av) | ((bv == av) & (bi < ai))
    minv = jnp.where(sw, bv, av)
    mini = jnp.where(sw, bi, ai)
    maxv = jnp.where(sw, av, bv)
    maxi = jnp.where(sw, ai, bi)
    return minv, mini, maxv, maxi


def _bitonic_sort(sv, si):
    m = len(sv)
    k = 2
    while k <= m:
        j = k // 2
        while j >= 1:
            for i in range(m):
                l = i ^ j
                if l > i:
                    a, b = (i, l) if (i & k) == 0 else (l, i)
                    sv[a], si[a], sv[b], si[b] = _lexswap(
                        sv[a], si[a], sv[b], si[b])
            j //= 2
        k *= 2


def _merge_low8(av, ai, bv, bi):
    mv, mi = [], []
    for i in range(_T):
        bv_r, bi_r = bv[_T - 1 - i], bi[_T - 1 - i]
        sw = (bv_r < av[i]) | ((bv_r == av[i]) & (bi_r < ai[i]))
        mv.append(jnp.where(sw, bv_r, av[i]))
        mi.append(jnp.where(sw, bi_r, ai[i]))
    j = _T // 2
    while j >= 1:
        for i in range(_T):
            if (i & j) == 0 and (i + j) < _T:
                mv[i], mi[i], mv[i + j], mi[i + j] = _lexswap(
                    mv[i], mi[i], mv[i + j], mi[i + j])
        j //= 2
    return mv, mi


def _select_kernel(tok_ref, ptsT_ref, d_ref, i_ref, ds_ref, is_ref):
    qt = tok_ref.shape[0]
    n = ptsT_ref.shape[1]
    nch = n // _CH
    t = tok_ref[...]
    tx = t[:, 0:1]
    ty = t[:, 1:2]
    tz = t[:, 2:3]
    iota128 = lax.broadcasted_iota(jnp.int32, (qt, 128), 1)
    nsl = _CH // 128

    def dist_chunk(off):
        px = ptsT_ref[0:1, pl.ds(off, _CH)]
        py = ptsT_ref[1:2, pl.ds(off, _CH)]
        pz = ptsT_ref[2:3, pl.ds(off, _CH)]
        dx = tx - px
        dy = ty - py
        dz = tz - pz
        d2 = (dx * dx + dy * dy) + dz * dz
        dist = jnp.sqrt(jnp.maximum(d2, 1e-12))
        return jnp.where(dist <= _RADIUS, dist, jnp.inf)

    def chunk_sort(c, carry):
        cv = list(carry[0])
        cix = list(carry[1])
        off = pl.multiple_of(c * _CH, _CH)
        v = dist_chunk(off)
        sv = [v[:, tt * 128:(tt + 1) * 128] for tt in range(nsl)]
        si = [iota128 + (off + tt * 128) for tt in range(nsl)]
        a_v, a_i = sv[:_T], si[:_T]
        b_v, b_i = sv[_T:], si[_T:]
        _bitonic_sort(a_v, a_i)
        _bitonic_sort(b_v, b_i)
        mv, mi = _merge_low8(a_v, a_i, b_v, b_i)
        mv, mi = _merge_low8(cv, cix, mv, mi)
        return (tuple(mv), tuple(mi))

    init_cv = tuple(jnp.full((qt, 128), jnp.inf, jnp.float32) for _ in range(_T))
    init_ci = tuple(jnp.full((qt, 128), n, jnp.int32) for _ in range(_T))
    cv, cix = lax.fori_loop(0, nch, chunk_sort, (init_cv, init_ci))
    cand0 = jnp.concatenate(cv, axis=1)
    candi = jnp.concatenate(cix, axis=1)

    def extract(j, carry):
        cand, cnt = carry
        cm = jnp.min(cand, axis=1, keepdims=True)
        eqm = cand == cm
        ci = jnp.min(jnp.where(eqm, candi, n), axis=1, keepdims=True)
        ds_ref[pl.ds(j, 1), :] = cm.T
        is_ref[pl.ds(j, 1), :] = ci.T
        cand = jnp.where(eqm & (candi == ci), jnp.inf, cand)
        hit = (iota128 == (ci & 127)) & (cm <= _RADIUS)
        cnt = cnt + hit.astype(jnp.int32)
        return (cand, cnt)

    _, cnt = lax.fori_loop(
        0, _K, extract, (cand0, jnp.zeros((qt, 128), jnp.int32)))

    overflow = jnp.max(cnt) >= _T
    iota_ch = lax.broadcasted_iota(jnp.int32, (qt, _CH), 1)

    @pl.when(overflow)
    def _slow():
        def extract_flat(j, carry):
            mprev, iprev = carry

            def chunk(c, mc):
                m, ii = mc
                off = pl.multiple_of(c * _CH, _CH)
                v = dist_chunk(off)
                g = iota_ch + c * _CH
                act = (v > mprev) | ((v == mprev) & (g > iprev))
                vm = jnp.where(act, v, jnp.inf)
                cm = jnp.min(vm, axis=1, keepdims=True)
                ci = jnp.min(jnp.where(vm == cm, g, n), axis=1, keepdims=True)
                take = (cm < m) | ((cm == m) & (ci < ii))
                ii = jnp.where(take, ci, ii)
                m = jnp.where(take, cm, m)
                return (m, ii)

            m, idx = lax.fori_loop(
                0, nch, chunk,
                (jnp.full((qt, 1), jnp.inf, jnp.float32),
                 jnp.full((qt, 1), n, jnp.int32)))
            ds_ref[pl.ds(j, 1), :] = m.T
            is_ref[pl.ds(j, 1), :] = idx.T
            return (m, idx)

        lax.fori_loop(
            0, _K, extract_flat,
            (jnp.full((qt, 1), -jnp.inf, jnp.float32),
             jnp.full((qt, 1), -1, jnp.int32)))

    d = ds_ref[...].T
    ii = is_ref[...].T
    valid = d <= _RADIUS
    ii = jnp.where(valid, ii, ii[:, 0:1])
    d = jnp.where(valid, d, d[:, 0:1])
    d = jnp.where(d <= _RADIUS, d, 0.0)
    d_ref[...] = d
    i_ref[...] = ii


def _tc_select(ptsT, tok):
    nt = tok.shape[0]
    n = ptsT.shape[1]
    return pl.pallas_call(
        _select_kernel,
        grid=(nt // _QT,),
        in_specs=[
            pl.BlockSpec((_QT, 3), lambda i: (i, 0)),
            pl.BlockSpec((3, n), lambda i: (0, 0)),
        ],
        out_specs=[
            pl.BlockSpec((_QT, _K), lambda i: (i, 0)),
            pl.BlockSpec((_QT, _K), lambda i: (i, 0)),
        ],
        out_shape=[
            jax.ShapeDtypeStruct((nt, _K), jnp.float32),
            jax.ShapeDtypeStruct((nt, _K), jnp.int32),
        ],
        scratch_shapes=[
            pltpu.VMEM((_K, _QT), jnp.float32),
            pltpu.VMEM((_K, _QT), jnp.int32),
        ],
    )(tok, ptsT)


def _sc_feats(pts_flat, inds_flat, d_flat, tok_flat, nt):
    qpw = nt // _NW
    groups = qpw // 16
    tokpad = ((qpw * 3 + 127) // 128) * 128
    npts3 = pts_flat.shape[0]
    mesh = plsc.VectorSubcoreMesh(core_axis_name="c", subcore_axis_name="s")

    @functools.partial(
        pl.kernel,
        mesh=mesh,
        out_type=jax.ShapeDtypeStruct((nt * _K * 7,), jnp.float32),
        compiler_params=pltpu.CompilerParams(needs_layout_passes=False),
        scratch_types=[
            pltpu.VMEM((npts3,), jnp.float32),
            pltpu.VMEM((qpw * _K,), jnp.int32),
            pltpu.VMEM((qpw * _K,), jnp.float32),
            pltpu.VMEM((tokpad,), jnp.float32),
            pltpu.VMEM((qpw * _K * 7,), jnp.float32),
        ],
    )
    def k(pts_hbm, inds_hbm, dd_hbm, tok_hbm, out_hbm,
          pts_v, inds_v, dd_v, tok_v, feat_v):
        wid = lax.axis_index("s") * 2 + lax.axis_index("c")
        qb = wid * qpw
        pltpu.sync_copy(pts_hbm, pts_v)
        pltpu.sync_copy(inds_hbm.at[pl.ds(qb * _K, qpw * _K)], inds_v)
        pltpu.sync_copy(dd_hbm.at[pl.ds(qb * _K, qpw * _K)], dd_v)
        pltpu.sync_copy(tok_hbm.at[pl.ds(qb * 3, qpw * 3)], tok_v.at[pl.ds(0, qpw * 3)])
        lane = lax.iota(jnp.int32, 16)

        def group_body(g, carry):
            q = g * 16 + lane
            qx = plsc.load_gather(tok_v, [q * 3])
            qy = plsc.load_gather(tok_v, [q * 3 + 1])
            qz = plsc.load_gather(tok_v, [q * 3 + 2])
            for kk in range(_K):
                src = q * _K + kk
                idx = plsc.load_gather(inds_v, [src])
                dd = plsc.load_gather(dd_v, [src])
                px = plsc.load_gather(pts_v, [idx * 3])
                py = plsc.load_gather(pts_v, [idx * 3 + 1])
                pz = plsc.load_gather(pts_v, [idx * 3 + 2])
                rx = px - qx
                ry = py - qy
                rz = pz - qz
                base = q * (_K * 7) + kk * 7
                plsc.store_scatter(feat_v, [base], dd)
                plsc.store_scatter(feat_v, [base + 1], rx)
                plsc.store_scatter(feat_v, [base + 2], ry)
                plsc.store_scatter(feat_v, [base + 3], rz)
                plsc.store_scatter(feat_v, [base + 4], rx)
                plsc.store_scatter(feat_v, [base + 5], ry)
                plsc.store_scatter(feat_v, [base + 6], rz)
            return carry

        lax.fori_loop(0, groups, group_body, 0)
        pltpu.sync_copy(feat_v, out_hbm.at[pl.ds(qb * (_K * 7), qpw * (_K * 7))])

    return k(pts_flat, inds_flat, d_flat, tok_flat)


def _mlp_kernel(x_ref, w1_ref, b1_ref, w2_ref, b2_ref, w3_ref, b3_ref, o_ref):
    x = x_ref[...]
    h = jnp.dot(x, w1_ref[...], preferred_element_type=jnp.float32) + b1_ref[...]
    h = jax.nn.gelu(h)
    h = jnp.dot(h, w2_ref[...], preferred_element_type=jnp.float32) + b2_ref[...]
    h = jax.nn.gelu(h)
    o_ref[...] = jnp.dot(h, w3_ref[...], preferred_element_type=jnp.float32) + b3_ref[...]


def _tc_mlp(x, W1, b1, W2, b2, W3, b3):
    nt, fin = x.shape
    h1 = W1.shape[1]
    h2 = W2.shape[1]
    dout = W3.shape[1]
    return pl.pallas_call(
        _mlp_kernel,
        grid=(nt // _MT,),
        in_specs=[
            pl.BlockSpec((_MT, fin), lambda i: (i, 0)),
            pl.BlockSpec((fin, h1), lambda i: (0, 0)),
            pl.BlockSpec((1, h1), lambda i: (0, 0)),
            pl.BlockSpec((h1, h2), lambda i: (0, 0)),
            pl.BlockSpec((1, h2), lambda i: (0, 0)),
            pl.BlockSpec((h2, dout), lambda i: (0, 0)),
            pl.BlockSpec((1, dout), lambda i: (0, 0)),
        ],
        out_specs=pl.BlockSpec((_MT, dout), lambda i: (i, 0)),
        out_shape=jax.ShapeDtypeStruct((nt, dout), jnp.float32),
    )(x, W1, b1[None, :], W2, b2[None, :], W3, b3[None, :])


def kernel(geometry, tokens, W1, b1, W2, b2, W3, b3):
    B, _, _ = geometry.shape
    nt = tokens.shape[1]
    outs = []
    for b in range(B):
        pts = geometry[b]
        tok = tokens[b]
        d, ii = _tc_select(jnp.transpose(pts), tok)
        feats_flat = _sc_feats(pts.reshape(-1), ii.reshape(-1),
                               d.reshape(-1), tok.reshape(-1), nt)
        feats = feats_flat.reshape(nt, _K * 7)
        outs.append(_tc_mlp(feats, W1, b1, W2, b2, W3, b3))
    return jnp.stack(outs, axis=0)


# speedup vs baseline: 1.0296x; 1.0296x over previous
import functools
import jax
import jax.numpy as jnp
from jax import lax
from jax.experimental import pallas as pl
from jax.experimental.pallas import tpu as pltpu
from jax.experimental.pallas import tpu_sc as plsc

_RADIUS = 0.1
_K = 32
_QT = 128
_MT = 256
_NW = 32


_CH = 2048
_T = 8
_TX = 6


def _lexswap(av, ai, bv, bi):
    sw = (bv < av) | ((bv == av) & (bi < ai))
    minv = jnp.where(sw, bv, av)
    mini = jnp.where(sw, bi, ai)
    maxv = jnp.where(sw, av, bv)
    maxi = jnp.where(sw, ai, bi)
    return minv, mini, maxv, maxi


def _bitonic_sort(sv, si):
    m = len(sv)
    k = 2
    while k <= m:
        j = k // 2
        while j >= 1:
            for i in range(m):
                l = i ^ j
                if l > i:
                    a, b = (i, l) if (i & k) == 0 else (l, i)
                    sv[a], si[a], sv[b], si[b] = _lexswap(
                        sv[a], si[a], sv[b], si[b])
            j //= 2
        k *= 2


def _merge_low8(av, ai, bv, bi):
    mv, mi = [], []
    for i in range(_T):
        bv_r, bi_r = bv[_T - 1 - i], bi[_T - 1 - i]
        sw = (bv_r < av[i]) | ((bv_r == av[i]) & (bi_r < ai[i]))
        mv.append(jnp.where(sw, bv_r, av[i]))
        mi.append(jnp.where(sw, bi_r, ai[i]))
    j = _T // 2
    while j >= 1:
        for i in range(_T):
            if (i & j) == 0 and (i + j) < _T:
                mv[i], mi[i], mv[i + j], mi[i + j] = _lexswap(
                    mv[i], mi[i], mv[i + j], mi[i + j])
        j //= 2
    return mv, mi


def _select_kernel(tok_ref, ptsT_ref, d_ref, i_ref, ds_ref, is_ref):
    qt = tok_ref.shape[0]
    n = ptsT_ref.shape[1]
    nch = n // _CH
    t = tok_ref[...]
    tx = t[:, 0:1]
    ty = t[:, 1:2]
    tz = t[:, 2:3]
    iota128 = lax.broadcasted_iota(jnp.int32, (qt, 128), 1)
    nsl = _CH // 128

    def dist_chunk(off):
        px = ptsT_ref[0:1, pl.ds(off, _CH)]
        py = ptsT_ref[1:2, pl.ds(off, _CH)]
        pz = ptsT_ref[2:3, pl.ds(off, _CH)]
        dx = tx - px
        dy = ty - py
        dz = tz - pz
        d2 = (dx * dx + dy * dy) + dz * dz
        dist = jnp.sqrt(jnp.maximum(d2, 1e-12))
        return jnp.where(dist <= _RADIUS, dist, jnp.inf)

    def chunk_sort(c, carry):
        cv = list(carry[0])
        cix = list(carry[1])
        off = pl.multiple_of(c * _CH, _CH)
        v = dist_chunk(off)
        sv = [v[:, tt * 128:(tt + 1) * 128] for tt in range(nsl)]
        si = [iota128 + (off + tt * 128) for tt in range(nsl)]
        a_v, a_i = sv[:_T], si[:_T]
        b_v, b_i = sv[_T:], si[_T:]
        _bitonic_sort(a_v, a_i)
        _bitonic_sort(b_v, b_i)
        mv, mi = _merge_low8(a_v, a_i, b_v, b_i)
        mv, mi = _merge_low8(cv, cix, mv, mi)
        return (tuple(mv), tuple(mi))

    init_cv = tuple(jnp.full((qt, 128), jnp.inf, jnp.float32) for _ in range(_T))
    init_ci = tuple(jnp.full((qt, 128), n, jnp.int32) for _ in range(_T))
    cv, cix = lax.fori_loop(0, nch, chunk_sort, (init_cv, init_ci))
    cand = jnp.concatenate(cv[:_TX], axis=1)
    candi = jnp.concatenate(cix[:_TX], axis=1)

    def extract(j, carry):
        mprev, iprev, cnt = carry
        act = (cand > mprev) | ((cand == mprev) & (candi > iprev))
        vm = jnp.where(act, cand, jnp.inf)
        cm = jnp.min(vm, axis=1, keepdims=True)
        ci = jnp.min(jnp.where(vm == cm, candi, n), axis=1, keepdims=True)
        ds_ref[pl.ds(j, 1), :] = cm.T
        is_ref[pl.ds(j, 1), :] = ci.T
        hit = (iota128 == (ci & 127)) & (cm <= _RADIUS)
        cnt = cnt + hit.astype(jnp.int32)
        return (cm, ci, cnt)

    _, _, cnt = lax.fori_loop(
        0, _K, extract,
        (jnp.full((qt, 1), -jnp.inf, jnp.float32),
         jnp.full((qt, 1), -1, jnp.int32),
         jnp.zeros((qt, 128), jnp.int32)))

    overflow = jnp.max(cnt) >= _TX
    iota_ch = lax.broadcasted_iota(jnp.int32, (qt, _CH), 1)

    @pl.when(overflow)
    def _slow():
        def extract_flat(j, carry):
            mprev, iprev = carry

            def chunk(c, mc):
                m, ii = mc
                off = pl.multiple_of(c * _CH, _CH)
                v = dist_chunk(off)
                g = iota_ch + c * _CH
                act = (v > mprev) | ((v == mprev) & (g > iprev))
                vm = jnp.where(act, v, jnp.inf)
                cm = jnp.min(vm, axis=1, keepdims=True)
                ci = jnp.min(jnp.where(vm == cm, g, n), axis=1, keepdims=True)
                take = (cm < m) | ((cm == m) & (ci < ii))
                ii = jnp.where(take, ci, ii)
                m = jnp.where(take, cm, m)
                return (m, ii)

            m, idx = lax.fori_loop(
                0, nch, chunk,
                (jnp.full((qt, 1), jnp.inf, jnp.float32),
                 jnp.full((qt, 1), n, jnp.int32)))
            ds_ref[pl.ds(j, 1), :] = m.T
            is_ref[pl.ds(j, 1), :] = idx.T
            return (m, idx)

        lax.fori_loop(
            0, _K, extract_flat,
            (jnp.full((qt, 1), -jnp.inf, jnp.float32),
             jnp.full((qt, 1), -1, jnp.int32)))

    d = ds_ref[...].T
    ii = is_ref[...].T
    valid = d <= _RADIUS
    ii = jnp.where(valid, ii, ii[:, 0:1])
    d = jnp.where(valid, d, d[:, 0:1])
    d = jnp.where(d <= _RADIUS, d, 0.0)
    d_ref[...] = d
    i_ref[...] = ii


def _tc_select(ptsT, tok):
    nt = tok.shape[0]
    n = ptsT.shape[1]
    return pl.pallas_call(
        _select_kernel,
        grid=(nt // _QT,),
        in_specs=[
            pl.BlockSpec((_QT, 3), lambda i: (i, 0)),
            pl.BlockSpec((3, n), lambda i: (0, 0)),
        ],
        out_specs=[
            pl.BlockSpec((_QT, _K), lambda i: (i, 0)),
            pl.BlockSpec((_QT, _K), lambda i: (i, 0)),
        ],
        out_shape=[
            jax.ShapeDtypeStruct((nt, _K), jnp.float32),
            jax.ShapeDtypeStruct((nt, _K), jnp.int32),
        ],
        scratch_shapes=[
            pltpu.VMEM((_K, _QT), jnp.float32),
            pltpu.VMEM((_K, _QT), jnp.int32),
        ],
    )(tok, ptsT)


def _sc_feats(pts_flat, inds_flat, d_flat, tok_flat, nt):
    qpw = nt // _NW
    groups = qpw // 16
    tokpad = ((qpw * 3 + 127) // 128) * 128
    npts3 = pts_flat.shape[0]
    mesh = plsc.VectorSubcoreMesh(core_axis_name="c", subcore_axis_name="s")

    @functools.partial(
        pl.kernel,
        mesh=mesh,
        out_type=jax.ShapeDtypeStruct((nt * _K * 7,), jnp.float32),
        compiler_params=pltpu.CompilerParams(needs_layout_passes=False),
        scratch_types=[
            pltpu.VMEM((npts3,), jnp.float32),
            pltpu.VMEM((qpw * _K,), jnp.int32),
            pltpu.VMEM((qpw * _K,), jnp.float32),
            pltpu.VMEM((tokpad,), jnp.float32),
            pltpu.VMEM((qpw * _K * 7,), jnp.float32),
        ],
    )
    def k(pts_hbm, inds_hbm, dd_hbm, tok_hbm, out_hbm,
          pts_v, inds_v, dd_v, tok_v, feat_v):
        wid = lax.axis_index("s") * 2 + lax.axis_index("c")
        qb = wid * qpw
        pltpu.sync_copy(pts_hbm, pts_v)
        pltpu.sync_copy(inds_hbm.at[pl.ds(qb * _K, qpw * _K)], inds_v)
        pltpu.sync_copy(dd_hbm.at[pl.ds(qb * _K, qpw * _K)], dd_v)
        pltpu.sync_copy(tok_hbm.at[pl.ds(qb * 3, qpw * 3)], tok_v.at[pl.ds(0, qpw * 3)])
        lane = lax.iota(jnp.int32, 16)

        def group_body(g, carry):
            q = g * 16 + lane
            qx = plsc.load_gather(tok_v, [q * 3])
            qy = plsc.load_gather(tok_v, [q * 3 + 1])
            qz = plsc.load_gather(tok_v, [q * 3 + 2])
            for kk in range(_K):
                src = q * _K + kk
                idx = plsc.load_gather(inds_v, [src])
                dd = plsc.load_gather(dd_v, [src])
                px = plsc.load_gather(pts_v, [idx * 3])
                py = plsc.load_gather(pts_v, [idx * 3 + 1])
                pz = plsc.load_gather(pts_v, [idx * 3 + 2])
                rx = px - qx
                ry = py - qy
                rz = pz - qz
                base = q * (_K * 7) + kk * 7
                plsc.store_scatter(feat_v, [base], dd)
                plsc.store_scatter(feat_v, [base + 1], rx)
                plsc.store_scatter(feat_v, [base + 2], ry)
                plsc.store_scatter(feat_v, [base + 3], rz)
                plsc.store_scatter(feat_v, [base + 4], rx)
                plsc.store_scatter(feat_v, [base + 5], ry)
                plsc.store_scatter(feat_v, [base + 6], rz)
            return carry

        lax.fori_loop(0, groups, group_body, 0)
        pltpu.sync_copy(feat_v, out_hbm.at[pl.ds(qb * (_K * 7), qpw * (_K * 7))])

    return k(pts_flat, inds_flat, d_flat, tok_flat)


def _mlp_kernel(x_ref, w1_ref, b1_ref, w2_ref, b2_ref, w3_ref, b3_ref, o_ref):
    x = x_ref[...]
    h = jnp.dot(x, w1_ref[...], preferred_element_type=jnp.float32) + b1_ref[...]
    h = jax.nn.gelu(h)
    h = jnp.dot(h, w2_ref[...], preferred_element_type=jnp.float32) + b2_ref[...]
    h = jax.nn.gelu(h)
    o_ref[...] = jnp.dot(h, w3_ref[...], preferred_element_type=jnp.float32) + b3_ref[...]


def _tc_mlp(x, W1, b1, W2, b2, W3, b3):
    nt, fin = x.shape
    h1 = W1.shape[1]
    h2 = W2.shape[1]
    dout = W3.shape[1]
    return pl.pallas_call(
        _mlp_kernel,
        grid=(nt // _MT,),
        in_specs=[
            pl.BlockSpec((_MT, fin), lambda i: (i, 0)),
            pl.BlockSpec((fin, h1), lambda i: (0, 0)),
            pl.BlockSpec((1, h1), lambda i: (0, 0)),
            pl.BlockSpec((h1, h2), lambda i: (0, 0)),
            pl.BlockSpec((1, h2), lambda i: (0, 0)),
            pl.BlockSpec((h2, dout), lambda i: (0, 0)),
            pl.BlockSpec((1, dout), lambda i: (0, 0)),
        ],
        out_specs=pl.BlockSpec((_MT, dout), lambda i: (i, 0)),
        out_shape=jax.ShapeDtypeStruct((nt, dout), jnp.float32),
    )(x, W1, b1[None, :], W2, b2[None, :], W3, b3[None, :])


def kernel(geometry, tokens, W1, b1, W2, b2, W3, b3):
    B, _, _ = geometry.shape
    nt = tokens.shape[1]
    outs = []
    for b in range(B):
        pts = geometry[b]
        tok = tokens[b]
        d, ii = _tc_select(jnp.transpose(pts), tok)
        feats_flat = _sc_feats(pts.reshape(-1), ii.reshape(-1),
                               d.reshape(-1), tok.reshape(-1), nt)
        feats = feats_flat.reshape(nt, _K * 7)
        outs.append(_tc_mlp(feats, W1, b1, W2, b2, W3, b3))
    return jnp.stack(outs, axis=0)


# speedup vs baseline: 1.0757x; 1.0448x over previous
import functools
import jax
import jax.numpy as jnp
from jax import lax
from jax.experimental import pallas as pl
from jax.experimental.pallas import tpu as pltpu
from jax.experimental.pallas import tpu_sc as plsc

_RADIUS = 0.1
_K = 32
_QT = 128
_MT = 256
_NW = 32


_CH = 2048
_T = 8
_TX = 6


def _lexswap(av, ai, bv, bi):
    sw = (bv < av) | ((bv == av) & (bi < ai))
    minv = jnp.where(sw, bv, av)
    mini = jnp.where(sw, bi, ai)
    maxv = jnp.where(sw, av, bv)
    maxi = jnp.where(sw, ai, bi)
    return minv, mini, maxv, maxi


def _bitonic_sort(sv, si):
    m = len(sv)
    k = 2
    while k <= m:
        j = k // 2
        while j >= 1:
            for i in range(m):
                l = i ^ j
                if l > i:
                    a, b = (i, l) if (i & k) == 0 else (l, i)
                    sv[a], si[a], sv[b], si[b] = _lexswap(
                        sv[a], si[a], sv[b], si[b])
            j //= 2
        k *= 2


def _merge_low8(av, ai, bv, bi):
    mv, mi = [], []
    for i in range(_T):
        bv_r, bi_r = bv[_T - 1 - i], bi[_T - 1 - i]
        sw = (bv_r < av[i]) | ((bv_r == av[i]) & (bi_r < ai[i]))
        mv.append(jnp.where(sw, bv_r, av[i]))
        mi.append(jnp.where(sw, bi_r, ai[i]))
    j = _T // 2
    while j >= 1:
        for i in range(_T):
            if (i & j) == 0 and (i + j) < _T:
                mv[i], mi[i], mv[i + j], mi[i + j] = _lexswap(
                    mv[i], mi[i], mv[i + j], mi[i + j])
        j //= 2
    return mv, mi


def _select_kernel(tok_ref, ptsT_ref, d_ref, i_ref, vals_ref, ds_ref, is_ref):
    qt = tok_ref.shape[0]
    n = ptsT_ref.shape[1]
    nch = n // _CH
    t = tok_ref[...]
    tx = t[:, 0:1]
    ty = t[:, 1:2]
    tz = t[:, 2:3]
    iota128 = lax.broadcasted_iota(jnp.int32, (qt, 128), 1)
    nsl = _CH // 128

    def init_chunk(c, carry):
        off = pl.multiple_of(c * _CH, _CH)
        px = ptsT_ref[0:1, pl.ds(off, _CH)]
        py = ptsT_ref[1:2, pl.ds(off, _CH)]
        pz = ptsT_ref[2:3, pl.ds(off, _CH)]
        dx = tx - px
        dy = ty - py
        dz = tz - pz
        d2 = (dx * dx + dy * dy) + dz * dz
        dist = jnp.sqrt(jnp.maximum(d2, 1e-12))
        vals_ref[:, pl.ds(off, _CH)] = jnp.where(dist <= _RADIUS, dist, jnp.inf)
        return carry

    lax.fori_loop(0, nch, init_chunk, 0)

    def chunk_sort(c, carry):
        cv = list(carry[0])
        cix = list(carry[1])
        off = pl.multiple_of(c * _CH, _CH)
        v = vals_ref[:, pl.ds(off, _CH)]
        sv = [v[:, tt * 128:(tt + 1) * 128] for tt in range(nsl)]
        si = [iota128 + (off + tt * 128) for tt in range(nsl)]
        _bitonic_sort(sv, si)
        mv, mi = _merge_low8(cv, cix, sv[:_T], si[:_T])
        return (tuple(mv), tuple(mi))

    init_cv = tuple(jnp.full((qt, 128), jnp.inf, jnp.float32) for _ in range(_T))
    init_ci = tuple(jnp.full((qt, 128), n, jnp.int32) for _ in range(_T))
    cv, cix = lax.fori_loop(0, nch, chunk_sort, (init_cv, init_ci))
    cand = jnp.concatenate(cv[:_TX], axis=1)
    candi = jnp.concatenate(cix[:_TX], axis=1)

    def extract(j, carry):
        mprev, iprev, cnt = carry
        act = (cand > mprev) | ((cand == mprev) & (candi > iprev))
        vm = jnp.where(act, cand, jnp.inf)
        cm = jnp.min(vm, axis=1, keepdims=True)
        ci = jnp.min(jnp.where(vm == cm, candi, n), axis=1, keepdims=True)
        ds_ref[pl.ds(j, 1), :] = cm.T
        is_ref[pl.ds(j, 1), :] = ci.T
        hit = (iota128 == (ci & 127)) & (cm <= _RADIUS)
        cnt = cnt + hit.astype(jnp.int32)
        return (cm, ci, cnt)

    _, _, cnt = lax.fori_loop(
        0, _K, extract,
        (jnp.full((qt, 1), -jnp.inf, jnp.float32),
         jnp.full((qt, 1), -1, jnp.int32),
         jnp.zeros((qt, 128), jnp.int32)))

    overflow = jnp.max(cnt) >= _TX
    iota_ch = lax.broadcasted_iota(jnp.int32, (qt, _CH), 1)

    @pl.when(overflow)
    def _slow():
        def extract_flat(j, carry):
            mprev, iprev = carry

            def chunk(c, mc):
                m, ii = mc
                off = pl.multiple_of(c * _CH, _CH)
                v = vals_ref[:, pl.ds(off, _CH)]
                g = iota_ch + c * _CH
                act = (v > mprev) | ((v == mprev) & (g > iprev))
                vm = jnp.where(act, v, jnp.inf)
                cm = jnp.min(vm, axis=1, keepdims=True)
                ci = jnp.min(jnp.where(vm == cm, g, n), axis=1, keepdims=True)
                take = (cm < m) | ((cm == m) & (ci < ii))
                ii = jnp.where(take, ci, ii)
                m = jnp.where(take, cm, m)
                return (m, ii)

            m, idx = lax.fori_loop(
                0, nch, chunk,
                (jnp.full((qt, 1), jnp.inf, jnp.float32),
                 jnp.full((qt, 1), n, jnp.int32)))
            ds_ref[pl.ds(j, 1), :] = m.T
            is_ref[pl.ds(j, 1), :] = idx.T
            return (m, idx)

        lax.fori_loop(
            0, _K, extract_flat,
            (jnp.full((qt, 1), -jnp.inf, jnp.float32),
             jnp.full((qt, 1), -1, jnp.int32)))

    d = ds_ref[...].T
    ii = is_ref[...].T
    valid = d <= _RADIUS
    ii = jnp.where(valid, ii, ii[:, 0:1])
    d = jnp.where(valid, d, d[:, 0:1])
    d = jnp.where(d <= _RADIUS, d, 0.0)
    d_ref[...] = d
    i_ref[...] = ii


def _tc_select(ptsT, tok):
    nt = tok.shape[0]
    n = ptsT.shape[1]
    return pl.pallas_call(
        _select_kernel,
        grid=(nt // _QT,),
        in_specs=[
            pl.BlockSpec((_QT, 3), lambda i: (i, 0)),
            pl.BlockSpec((3, n), lambda i: (0, 0)),
        ],
        out_specs=[
            pl.BlockSpec((_QT, _K), lambda i: (i, 0)),
            pl.BlockSpec((_QT, _K), lambda i: (i, 0)),
        ],
        out_shape=[
            jax.ShapeDtypeStruct((nt, _K), jnp.float32),
            jax.ShapeDtypeStruct((nt, _K), jnp.int32),
        ],
        scratch_shapes=[
            pltpu.VMEM((_QT, n), jnp.float32),
            pltpu.VMEM((_K, _QT), jnp.float32),
            pltpu.VMEM((_K, _QT), jnp.int32),
        ],
    )(tok, ptsT)


def _sc_feats(pts_flat, inds_flat, d_flat, tok_flat, nt):
    qpw = nt // _NW
    groups = qpw // 16
    tokpad = ((qpw * 3 + 127) // 128) * 128
    npts3 = pts_flat.shape[0]
    mesh = plsc.VectorSubcoreMesh(core_axis_name="c", subcore_axis_name="s")

    @functools.partial(
        pl.kernel,
        mesh=mesh,
        out_type=jax.ShapeDtypeStruct((nt * _K * 7,), jnp.float32),
        compiler_params=pltpu.CompilerParams(needs_layout_passes=False),
        scratch_types=[
            pltpu.VMEM((npts3,), jnp.float32),
            pltpu.VMEM((qpw * _K,), jnp.int32),
            pltpu.VMEM((qpw * _K,), jnp.float32),
            pltpu.VMEM((tokpad,), jnp.float32),
            pltpu.VMEM((qpw * _K * 7,), jnp.float32),
        ],
    )
    def k(pts_hbm, inds_hbm, dd_hbm, tok_hbm, out_hbm,
          pts_v, inds_v, dd_v, tok_v, feat_v):
        wid = lax.axis_index("s") * 2 + lax.axis_index("c")
        qb = wid * qpw
        pltpu.sync_copy(pts_hbm, pts_v)
        pltpu.sync_copy(inds_hbm.at[pl.ds(qb * _K, qpw * _K)], inds_v)
        pltpu.sync_copy(dd_hbm.at[pl.ds(qb * _K, qpw * _K)], dd_v)
        pltpu.sync_copy(tok_hbm.at[pl.ds(qb * 3, qpw * 3)], tok_v.at[pl.ds(0, qpw * 3)])
        lane = lax.iota(jnp.int32, 16)

        def group_body(g, carry):
            q = g * 16 + lane
            qx = plsc.load_gather(tok_v, [q * 3])
            qy = plsc.load_gather(tok_v, [q * 3 + 1])
            qz = plsc.load_gather(tok_v, [q * 3 + 2])
            for kk in range(_K):
                src = q * _K + kk
                idx = plsc.load_gather(inds_v, [src])
                dd = plsc.load_gather(dd_v, [src])
                px = plsc.load_gather(pts_v, [idx * 3])
                py = plsc.load_gather(pts_v, [idx * 3 + 1])
                pz = plsc.load_gather(pts_v, [idx * 3 + 2])
                rx = px - qx
                ry = py - qy
                rz = pz - qz
                base = q * (_K * 7) + kk * 7
                plsc.store_scatter(feat_v, [base], dd)
                plsc.store_scatter(feat_v, [base + 1], rx)
                plsc.store_scatter(feat_v, [base + 2], ry)
                plsc.store_scatter(feat_v, [base + 3], rz)
                plsc.store_scatter(feat_v, [base + 4], rx)
                plsc.store_scatter(feat_v, [base + 5], ry)
                plsc.store_scatter(feat_v, [base + 6], rz)
            return carry

        lax.fori_loop(0, groups, group_body, 0)
        pltpu.sync_copy(feat_v, out_hbm.at[pl.ds(qb * (_K * 7), qpw * (_K * 7))])

    return k(pts_flat, inds_flat, d_flat, tok_flat)


def _mlp_kernel(x_ref, w1_ref, b1_ref, w2_ref, b2_ref, w3_ref, b3_ref, o_ref):
    x = x_ref[...]
    h = jnp.dot(x, w1_ref[...], preferred_element_type=jnp.float32) + b1_ref[...]
    h = jax.nn.gelu(h)
    h = jnp.dot(h, w2_ref[...], preferred_element_type=jnp.float32) + b2_ref[...]
    h = jax.nn.gelu(h)
    o_ref[...] = jnp.dot(h, w3_ref[...], preferred_element_type=jnp.float32) + b3_ref[...]


def _tc_mlp(x, W1, b1, W2, b2, W3, b3):
    nt, fin = x.shape
    h1 = W1.shape[1]
    h2 = W2.shape[1]
    dout = W3.shape[1]
    return pl.pallas_call(
        _mlp_kernel,
        grid=(nt // _MT,),
        in_specs=[
            pl.BlockSpec((_MT, fin), lambda i: (i, 0)),
            pl.BlockSpec((fin, h1), lambda i: (0, 0)),
            pl.BlockSpec((1, h1), lambda i: (0, 0)),
            pl.BlockSpec((h1, h2), lambda i: (0, 0)),
            pl.BlockSpec((1, h2), lambda i: (0, 0)),
            pl.BlockSpec((h2, dout), lambda i: (0, 0)),
            pl.BlockSpec((1, dout), lambda i: (0, 0)),
        ],
        out_specs=pl.BlockSpec((_MT, dout), lambda i: (i, 0)),
        out_shape=jax.ShapeDtypeStruct((nt, dout), jnp.float32),
    )(x, W1, b1[None, :], W2, b2[None, :], W3, b3[None, :])


def kernel(geometry, tokens, W1, b1, W2, b2, W3, b3):
    B, _, _ = geometry.shape
    nt = tokens.shape[1]
    outs = []
    for b in range(B):
        pts = geometry[b]
        tok = tokens[b]
        d, ii = _tc_select(jnp.transpose(pts), tok)
        feats_flat = _sc_feats(pts.reshape(-1), ii.reshape(-1),
                               d.reshape(-1), tok.reshape(-1), nt)
        feats = feats_flat.reshape(nt, _K * 7)
        outs.append(_tc_mlp(feats, W1, b1, W2, b2, W3, b3))
    return jnp.stack(outs, axis=0)
